# baseline (speedup 1.0000x reference)
import jax
import jax.numpy as jnp
from jax.experimental import pallas as pl
from jax.experimental.pallas import tpu as pltpu

_BM = 1024
_NBUF = 4
_NCHUNK = 7


def _dot(a, b):
    return jax.lax.dot_general(
        a, b, dimension_numbers=(((1,), (0,)), ((), ())),
        preferred_element_type=jnp.float32)


def _make_body(nsteps):
    def body(x_hbm, w1_ref, b1_ref, w2_ref, b2_ref, out_hbm,
             xbuf, obuf, insems, outsems):
        def in_cps(i):
            cps = []
            for c in range(_NCHUNK):
                cps.append(pltpu.make_async_copy(
                    x_hbm.at[pl.ds(i * _BM, _BM), pl.ds(c * 128, 128)],
                    xbuf.at[i % _NBUF, :, pl.ds(c * 128, 128)],
                    insems.at[i % _NBUF, c]))
            return cps

        def out_cp(i):
            return pltpu.make_async_copy(
                obuf.at[i % _NBUF],
                out_hbm.at[pl.ds(i * _BM, _BM), :],
                outsems.at[i % _NBUF])

        w1 = w1_ref[...].astype(jnp.bfloat16)
        b1 = b1_ref[...]
        w2 = w2_ref[...].astype(jnp.bfloat16)
        b2 = b2_ref[...]

        for i in range(_NBUF - 1):
            for cp in in_cps(i):
                cp.start()
        for i in range(nsteps):
            for cp in in_cps(i):
                cp.wait()
            x = xbuf[i % _NBUF].astype(jnp.bfloat16)
            h = jnp.maximum(_dot(x, w1) + b1, 0.0)
            z = _dot(h.astype(jnp.bfloat16), w2) + b2
            if i >= _NBUF:
                out_cp(i - _NBUF).wait()
            obuf[i % _NBUF] = z
            out_cp(i).start()
            nxt = i + _NBUF - 1
            if nxt < nsteps:
                for cp in in_cps(nxt):
                    cp.start()
        for i in range(max(0, nsteps - _NBUF), nsteps):
            out_cp(i).wait()
    return body


def kernel(inputs, enc_w1, enc_b1, enc_w2, enc_b2,
           dec_w1, dec_b1, dec_w2, dec_b2, prior):
    del dec_w1, dec_b1, dec_w2, dec_b2, prior
    b, feat = inputs.shape
    hid = enc_w1.shape[1]
    code = enc_w2.shape[1]
    nsteps = b // _BM
    out = pl.pallas_call(
        _make_body(nsteps),
        in_specs=[
            pl.BlockSpec(memory_space=pl.ANY),
            pl.BlockSpec(memory_space=pltpu.VMEM),
            pl.BlockSpec(memory_space=pltpu.VMEM),
            pl.BlockSpec(memory_space=pltpu.VMEM),
            pl.BlockSpec(memory_space=pltpu.VMEM),
        ],
        out_specs=pl.BlockSpec(memory_space=pl.ANY),
        out_shape=jax.ShapeDtypeStruct((b, code), jnp.float32),
        scratch_shapes=[
            pltpu.VMEM((_NBUF, _BM, feat), jnp.float32),
            pltpu.VMEM((_NBUF, _BM, code), jnp.float32),
            pltpu.SemaphoreType.DMA((_NBUF, _NCHUNK)),
            pltpu.SemaphoreType.DMA((_NBUF,)),
        ],
    )(inputs, enc_w1, enc_b1.reshape(1, hid),
      enc_w2, enc_b2.reshape(1, code))
    return out


# speedup vs baseline: 1.0097x; 1.0097x over previous
import jax
import jax.numpy as jnp
from jax.experimental import pallas as pl
from jax.experimental.pallas import tpu as pltpu

_BM = 1024
_NBUF = 4


def _dot(a, b):
    return jax.lax.dot_general(
        a, b, dimension_numbers=(((1,), (0,)), ((), ())),
        preferred_element_type=jnp.float32)


def _make_body(nsteps):
    def body(x_hbm, w1_ref, b1_ref, w2_ref, b2_ref, out_hbm,
             xbuf, obuf, insems, outsems):
        def in_cp(i):
            return pltpu.make_async_copy(
                x_hbm.at[pl.ds(i * _BM, _BM), :],
                xbuf.at[i % _NBUF],
                insems.at[i % _NBUF])

        def out_cp(i):
            return pltpu.make_async_copy(
                obuf.at[i % _NBUF],
                out_hbm.at[pl.ds(i * _BM, _BM), :],
                outsems.at[i % _NBUF])

        w1 = w1_ref[...].astype(jnp.bfloat16)
        b1 = b1_ref[...]
        w2 = w2_ref[...].astype(jnp.bfloat16)
        b2 = b2_ref[...]

        for i in range(_NBUF - 1):
            in_cp(i).start()
        for i in range(nsteps):
            in_cp(i).wait()
            x = xbuf[i % _NBUF].astype(jnp.bfloat16)
            h = jnp.maximum(_dot(x, w1) + b1, 0.0)
            z = _dot(h.astype(jnp.bfloat16), w2) + b2
            if i >= _NBUF:
                out_cp(i - _NBUF).wait()
            obuf[i % _NBUF] = z
            out_cp(i).start()
            nxt = i + _NBUF - 1
            if nxt < nsteps:
                in_cp(nxt).start()
        for i in range(max(0, nsteps - _NBUF), nsteps):
            out_cp(i).wait()
    return body


def kernel(inputs, enc_w1, enc_b1, enc_w2, enc_b2,
           dec_w1, dec_b1, dec_w2, dec_b2, prior):
    del dec_w1, dec_b1, dec_w2, dec_b2, prior
    b, feat = inputs.shape
    hid = enc_w1.shape[1]
    code = enc_w2.shape[1]
    nsteps = b // _BM
    out = pl.pallas_call(
        _make_body(nsteps),
        in_specs=[
            pl.BlockSpec(memory_space=pl.ANY),
            pl.BlockSpec(memory_space=pltpu.VMEM),
            pl.BlockSpec(memory_space=pltpu.VMEM),
            pl.BlockSpec(memory_space=pltpu.VMEM),
            pl.BlockSpec(memory_space=pltpu.VMEM),
        ],
        out_specs=pl.BlockSpec(memory_space=pl.ANY),
        out_shape=jax.ShapeDtypeStruct((b, code), jnp.float32),
        scratch_shapes=[
            pltpu.VMEM((_NBUF, _BM, feat), jnp.float32),
            pltpu.VMEM((_NBUF, _BM, code), jnp.float32),
            pltpu.SemaphoreType.DMA((_NBUF,)),
            pltpu.SemaphoreType.DMA((_NBUF,)),
        ],
    )(inputs, enc_w1, enc_b1.reshape(1, hid),
      enc_w2, enc_b2.reshape(1, code))
    return out
